# baseline (speedup 1.0000x reference)
import functools

import jax
import jax.numpy as jnp
from jax import lax
from jax.experimental import pallas as pl
from jax.experimental.pallas import tpu as pltpu
from jax.experimental.pallas import tpu_sc as plsc

B = 16384
N_ROWS = 1000000
D = 32
V = 1000
L = 20
_G = 128
_RPG = _G // D

_RB = 1024
_NRB = 245
_NPACK = _NRB * _RB

_NC = 2
_NS = 16
_NW = _NC * _NS
_BPW = B // _NW
_CHUNK = 128
_NCHUNK = _BPW // _CHUNK


_NFULL = N_ROWS // _G
_TAIL = N_ROWS - _NFULL * _G
_NBUF = 8


def _relay_sc_body(xu, xi, tu, ti, ou, oi, sin, sout, sem_in, sem_out):
  wid = lax.axis_index("s") * _NC + lax.axis_index("c")
  iota16 = lax.iota(jnp.int32, 16)
  rowc = [(j * 16 + iota16) // _RPG for j in range(_G // 16)]
  colc = [((j * 16 + iota16) % _RPG) * D for j in range(_G // 16)]
  g_lo = (wid * _NFULL) // _NW
  g_hi = ((wid + 1) * _NFULL) // _NW
  count = g_hi - g_lo
  for x_t, t_t, o_t in ((xu, tu, ou), (xi, ti, oi)):

    def fire_in(g, slot):
      start = pl.multiple_of(g * _G, _G)
      pltpu.async_copy(x_t.at[:, pl.ds(start, _G)], sin.at[slot],
                       sem_in.at[slot])

    for k in range(_NBUF):
      fire_in(g_lo + k, k)

    def step(k, carry):
      s = lax.rem(k, _NBUF)
      srow = s * D
      pltpu.make_async_copy(x_t.at[:, pl.ds(0, _G)], sin.at[s],
                            sem_in.at[s]).wait()

      @pl.when(k >= _NBUF)
      def _():
        pltpu.make_async_copy(sout.at[pl.ds(0, D)], o_t.at[pl.ds(0, D)],
                              sem_out.at[s]).wait()

      rowv = [rc + srow for rc in rowc]

      def perm_d(d, c2):
        for j in range(_G // 16):
          v = sin[s, d, pl.ds(j * 16, 16)]
          plsc.store_scatter(sout, [rowv[j], colc[j] + d], v)
        return c2

      lax.fori_loop(0, D, perm_d, 0)
      row = (g_lo + k) * D
      pltpu.async_copy(sout.at[pl.ds(srow, D)], o_t.at[pl.ds(row, D)],
                       sem_out.at[s])

      @pl.when(k + _NBUF < count)
      def _():
        fire_in(g_lo + k + _NBUF, s)

      return carry

    lax.fori_loop(0, count, step, 0)
    for j in range(_NBUF):
      s = lax.rem(count - _NBUF + j, _NBUF)
      pltpu.make_async_copy(sout.at[pl.ds(0, D)], o_t.at[pl.ds(0, D)],
                            sem_out.at[s]).wait()

    @pl.when(wid == _NW - 1)
    def _():
      pltpu.sync_copy(t_t, sout.at[pl.ds(0, _TAIL // _RPG)])
      pltpu.sync_copy(sout.at[pl.ds(0, _TAIL // _RPG)],
                      o_t.at[pl.ds(_NFULL * D, _TAIL // _RPG)])


@functools.partial(
    pl.kernel,
    out_type=(jax.ShapeDtypeStruct((_NPACK, _G), jnp.float32),
              jax.ShapeDtypeStruct((_NPACK, _G), jnp.float32)),
    mesh=plsc.VectorSubcoreMesh(core_axis_name="c", subcore_axis_name="s"),
    scratch_types=[
        pltpu.VMEM((_NBUF, D, _G), jnp.float32),
        pltpu.VMEM((_NBUF * D, _G), jnp.float32),
        pltpu.SemaphoreType.DMA((_NBUF,)),
        pltpu.SemaphoreType.DMA((_NBUF,)),
    ],
    compiler_params=pltpu.CompilerParams(needs_layout_passes=False,
                                         disable_bounds_checks=True),
)
def _relay_call(*args):
  _relay_sc_body(*args)


def _sc_gather_body(user_packed, uids, item_packed, iids, ue_out, ie_out,
                    idx, rows, sem):
  wid = lax.axis_index("s") * _NC + lax.axis_index("c")
  base = wid * _BPW
  row0 = wid * _NCHUNK
  for table, ids, out in ((user_packed, uids, ue_out),
                          (item_packed, iids, ie_out)):
    pltpu.sync_copy(ids.at[pl.ds(row0, _NCHUNK)], idx)
    copies = []
    for j in range(_NCHUNK):
      copies.append(pltpu.async_copy(
          table.at[idx.at[j]], rows.at[pl.ds(j * _CHUNK, _CHUNK)], sem))
    for c in copies:
      c.wait()
    pltpu.sync_copy(rows, out.at[pl.ds(base, _BPW)])


@functools.partial(
    pl.kernel,
    out_type=(jax.ShapeDtypeStruct((B, _G), jnp.float32),
              jax.ShapeDtypeStruct((B, _G), jnp.float32)),
    mesh=plsc.VectorSubcoreMesh(core_axis_name="c", subcore_axis_name="s"),
    scratch_types=[
        pltpu.VMEM((_NCHUNK, _CHUNK), jnp.int32),
        pltpu.VMEM((_BPW, _G), jnp.float32),
        pltpu.SemaphoreType.DMA,
    ],
)
def _sc_gather(*args):
  _sc_gather_body(*args)


_TB = 1024


def _tc_body(lens_ref, uoff_ref, ioff_ref, ue_ref, ie_ref, wext_ref,
             logits_ref, mask_ref):
  col = lax.broadcasted_iota(jnp.int32, (_TB, _G), 1) // D
  mu = (col == uoff_ref[...]).astype(jnp.float32)
  mi = (col == ioff_ref[...]).astype(jnp.float32)
  e = jnp.concatenate([ue_ref[...] * mu, ie_ref[...] * mi], axis=1)
  logits_ref[...] = lax.dot_general(
      wext_ref[...], e, (((0,), (1,)), ((), ())),
      preferred_element_type=jnp.float32)
  io = lax.broadcasted_iota(jnp.int32, (_TB, L), 1)
  mask_ref[...] = io >= lens_ref[...]


_tc_call = pl.pallas_call(
    _tc_body,
    grid=(B // _TB,),
    in_specs=[
        pl.BlockSpec((_TB, 1), lambda i: (i, 0)),
        pl.BlockSpec((_TB, 1), lambda i: (i, 0)),
        pl.BlockSpec((_TB, 1), lambda i: (i, 0)),
        pl.BlockSpec((_TB, _G), lambda i: (i, 0)),
        pl.BlockSpec((_TB, _G), lambda i: (i, 0)),
        pl.BlockSpec((2 * _G, V), lambda i: (0, 0)),
    ],
    out_specs=[
        pl.BlockSpec((V, _TB), lambda i: (0, i)),
        pl.BlockSpec((_TB, L), lambda i: (i, 0)),
    ],
    out_shape=[
        jax.ShapeDtypeStruct((V, B), jnp.float32),
        jax.ShapeDtypeStruct((B, L), jnp.bool_),
    ],
)


def kernel(pos_attr_set, pos_attr_lens, neg_attr_set, neg_attr_lens,
           neg_attr_set_num, user_ids, item_ids, _, user_table, item_table,
           W_user, W_item):
  xu = user_table.T
  xi = item_table.T
  tail_u = user_table[_NFULL * _G:].reshape(_TAIL // _RPG, _G)
  tail_i = item_table[_NFULL * _G:].reshape(_TAIL // _RPG, _G)
  up, ip = _relay_call(xu, xi, tail_u, tail_i)
  uids = user_ids.astype(jnp.int32)
  iids = item_ids.astype(jnp.int32)
  uid_g = (uids >> 2).reshape(B // _CHUNK, _CHUNK)
  iid_g = (iids >> 2).reshape(B // _CHUNK, _CHUNK)
  ue, ie = _sc_gather(up, uid_g, ip, iid_g)
  wext = jnp.concatenate(
      [jnp.tile(W_user.T, (_RPG, 1)), jnp.tile(W_item.T, (_RPG, 1))], axis=0)
  logits_t, mask = _tc_call(
      pos_attr_lens.astype(jnp.int32).reshape(B, 1),
      (uids & 3).reshape(B, 1), (iids & 3).reshape(B, 1),
      ue, ie, wext)
  return (logits_t.T, mask)


# speedup vs baseline: 1.3757x; 1.3757x over previous
import functools

import jax
import jax.numpy as jnp
from jax import lax
from jax.experimental import pallas as pl
from jax.experimental.pallas import tpu as pltpu
from jax.experimental.pallas import tpu_sc as plsc

B = 16384
N_ROWS = 1000000
D = 32
V = 1000
L = 20
_G = 128
_RPG = _G // D
_NPACK = N_ROWS // _RPG

_NC = 2
_NS = 16
_NW = _NC * _NS
_BPW = B // _NW
_CHUNK = 128
_NCHUNK = _BPW // _CHUNK


def _sc_gather_body(user_packed, uids, item_packed, iids, ue_out, ie_out,
                    idx, rows, sem):
  wid = lax.axis_index("s") * _NC + lax.axis_index("c")
  base = wid * _BPW
  row0 = wid * _NCHUNK
  for table, ids, out in ((user_packed, uids, ue_out),
                          (item_packed, iids, ie_out)):
    pltpu.sync_copy(ids.at[pl.ds(row0, _NCHUNK)], idx)
    copies = []
    for j in range(_NCHUNK):
      copies.append(pltpu.async_copy(
          table.at[idx.at[j]], rows.at[pl.ds(j * _CHUNK, _CHUNK)], sem))
    for c in copies:
      c.wait()
    pltpu.sync_copy(rows, out.at[pl.ds(base, _BPW)])


@functools.partial(
    pl.kernel,
    out_type=(jax.ShapeDtypeStruct((B, _G), jnp.float32),
              jax.ShapeDtypeStruct((B, _G), jnp.float32)),
    mesh=plsc.VectorSubcoreMesh(core_axis_name="c", subcore_axis_name="s"),
    scratch_types=[
        pltpu.VMEM((_NCHUNK, _CHUNK), jnp.int32),
        pltpu.VMEM((_BPW, _G), jnp.float32),
        pltpu.SemaphoreType.DMA,
    ],
)
def _sc_gather(*args):
  _sc_gather_body(*args)


_TB = 1024


def _tc_body(lens_ref, uoff_ref, ioff_ref, ue_ref, ie_ref, wext_ref,
             logits_ref, mask_ref):
  col = lax.broadcasted_iota(jnp.int32, (_TB, _G), 1) // D
  mu = (col == uoff_ref[...]).astype(jnp.float32)
  mi = (col == ioff_ref[...]).astype(jnp.float32)
  e = jnp.concatenate([ue_ref[...] * mu, ie_ref[...] * mi], axis=1)
  logits_ref[...] = lax.dot_general(
      wext_ref[...], e, (((0,), (1,)), ((), ())),
      preferred_element_type=jnp.float32)
  io = lax.broadcasted_iota(jnp.int32, (_TB, L), 1)
  mask_ref[...] = io >= lens_ref[...]


_tc_call = pl.pallas_call(
    _tc_body,
    grid=(B // _TB,),
    in_specs=[
        pl.BlockSpec((_TB, 1), lambda i: (i, 0)),
        pl.BlockSpec((_TB, 1), lambda i: (i, 0)),
        pl.BlockSpec((_TB, 1), lambda i: (i, 0)),
        pl.BlockSpec((_TB, _G), lambda i: (i, 0)),
        pl.BlockSpec((_TB, _G), lambda i: (i, 0)),
        pl.BlockSpec((2 * _G, V), lambda i: (0, 0)),
    ],
    out_specs=[
        pl.BlockSpec((V, _TB), lambda i: (0, i)),
        pl.BlockSpec((_TB, L), lambda i: (i, 0)),
    ],
    out_shape=[
        jax.ShapeDtypeStruct((V, B), jnp.float32),
        jax.ShapeDtypeStruct((B, L), jnp.bool_),
    ],
)


def kernel(pos_attr_set, pos_attr_lens, neg_attr_set, neg_attr_lens,
           neg_attr_set_num, user_ids, item_ids, _, user_table, item_table,
           W_user, W_item):
  up = user_table.reshape(_NPACK, _G)
  ip = item_table.reshape(_NPACK, _G)
  uids = user_ids.astype(jnp.int32)
  iids = item_ids.astype(jnp.int32)
  uid_g = (uids >> 2).reshape(B // _CHUNK, _CHUNK)
  iid_g = (iids >> 2).reshape(B // _CHUNK, _CHUNK)
  ue, ie = _sc_gather(up, uid_g, ip, iid_g)
  wext = jnp.concatenate(
      [jnp.tile(W_user.T, (_RPG, 1)), jnp.tile(W_item.T, (_RPG, 1))], axis=0)
  logits_t, mask = _tc_call(
      pos_attr_lens.astype(jnp.int32).reshape(B, 1),
      (uids & 3).reshape(B, 1), (iids & 3).reshape(B, 1),
      ue, ie, wext)
  return (logits_t.T, mask)
